# initial kernel scaffold (unmeasured)
import functools

import jax
import jax.numpy as jnp
from jax import lax
from jax.experimental import pallas as pl
from jax.experimental.pallas import tpu as pltpu

N_DEV = 8
SQ = 2048
SQ_SHARD = 256
KV_USED = 2048
H_LOC = 8
DH = 128
D_MODEL = 1024
SCALE = 0.08838834764831843
NEG = -1e9


def _body(x_ref, wq_ref, k_ref, v_ref, wo_ref, out_ref,
          xg_ref, partial_ref, send_ref, comm_ref,
          ag_send_sems, ag_recv_sems, rs_send_sems, rs_recv_sems,
          copy_sem):
    i = lax.axis_index("i")
    right = lax.rem(i + 1, N_DEV)
    left = lax.rem(i + N_DEV - 1, N_DEV)

    barrier_sem = pltpu.get_barrier_semaphore()
    pl.semaphore_signal(barrier_sem, inc=1, device_id=(left,),
                        device_id_type=pl.DeviceIdType.MESH)
    pl.semaphore_signal(barrier_sem, inc=1, device_id=(right,),
                        device_id_type=pl.DeviceIdType.MESH)
    pl.semaphore_wait(barrier_sem, 2)

    own = pltpu.make_async_copy(x_ref, xg_ref.at[i], copy_sem)
    own.start()
    own.wait()

    for h in range(N_DEV - 1):
        o_send = lax.rem(i + N_DEV - h, N_DEV)
        rdma = pltpu.make_async_remote_copy(
            src_ref=xg_ref.at[o_send],
            dst_ref=xg_ref.at[o_send],
            send_sem=ag_send_sems.at[h],
            recv_sem=ag_recv_sems.at[h],
            device_id=(right,),
            device_id_type=pl.DeviceIdType.MESH,
        )
        rdma.start()
        rdma.wait()

    r_idx = lax.broadcasted_iota(jnp.int32, (SQ_SHARD, SQ_SHARD), 0)
    c_idx = lax.broadcasted_iota(jnp.int32, (SQ_SHARD, SQ_SHARD), 1)
    diag_keep = (c_idx // 64) <= (r_idx // 64)

    for o in range(N_DEV):
        L = SQ_SHARD * (o + 1)
        xq = xg_ref[o]

        def head_step(h, acc, L=L, xq=xq):
            q = jnp.dot(xq, wq_ref[h], preferred_element_type=jnp.float32)
            q = (q * SCALE).astype(jnp.bfloat16)
            k = k_ref[h, :L, :]
            s = lax.dot_general(q, k, (((1,), (1,)), ((), ())),
                                preferred_element_type=jnp.float32)
            s_diag = jnp.where(diag_keep, s[:, L - SQ_SHARD:], NEG)
            s = jnp.concatenate([s[:, :L - SQ_SHARD], s_diag], axis=1)
            m = jnp.max(s, axis=1, keepdims=True)
            w = jnp.exp(s - m)
            p = (w / jnp.sum(w, axis=1, keepdims=True)).astype(jnp.bfloat16)
            ctx = jnp.dot(p, v_ref[h, :L, :],
                          preferred_element_type=jnp.float32)
            return acc + jnp.dot(ctx.astype(jnp.bfloat16), wo_ref[h],
                                 preferred_element_type=jnp.float32)

        acc0 = jnp.zeros((SQ_SHARD, D_MODEL), jnp.float32)
        partial_ref[o] = lax.fori_loop(0, H_LOC, head_step, acc0)

    for t in range(N_DEV - 1):
        c_send = lax.rem(i + t + 1, N_DEV)
        if t == 0:
            send_ref[...] = partial_ref[c_send]
        else:
            send_ref[...] = comm_ref[t - 1] + partial_ref[c_send]
        rdma = pltpu.make_async_remote_copy(
            src_ref=send_ref,
            dst_ref=comm_ref.at[t],
            send_sem=rs_send_sems.at[t],
            recv_sem=rs_recv_sems.at[t],
            device_id=(left,),
            device_id_type=pl.DeviceIdType.MESH,
        )
        rdma.start()
        rdma.wait()

    out_ref[0] = comm_ref[N_DEV - 2] + partial_ref[i]


def kernel(x, Wq, K_ext, V_ext, Wo):
    i = lax.axis_index("i")

    x_b = x[0].astype(jnp.bfloat16)
    wq_b = Wq.reshape(D_MODEL, H_LOC, DH).transpose(1, 0, 2) \
             .astype(jnp.bfloat16)
    wo_b = Wo.reshape(H_LOC, DH, D_MODEL).astype(jnp.bfloat16)
    k_l = lax.dynamic_slice(
        K_ext, (0, 0, H_LOC * i, 0), (1, KV_USED, H_LOC, DH)
    )[0].transpose(1, 0, 2).astype(jnp.bfloat16)
    v_l = lax.dynamic_slice(
        V_ext, (0, 0, H_LOC * i, 0), (1, KV_USED, H_LOC, DH)
    )[0].transpose(1, 0, 2).astype(jnp.bfloat16)

    return pl.pallas_call(
        _body,
        out_shape=jax.ShapeDtypeStruct((1, SQ_SHARD, D_MODEL), jnp.float32),
        in_specs=[pl.BlockSpec(memory_space=pltpu.VMEM)] * 5,
        out_specs=pl.BlockSpec(memory_space=pltpu.VMEM),
        scratch_shapes=[
            pltpu.VMEM((N_DEV, SQ_SHARD, D_MODEL), jnp.bfloat16),
            pltpu.VMEM((N_DEV, SQ_SHARD, D_MODEL), jnp.float32),
            pltpu.VMEM((SQ_SHARD, D_MODEL), jnp.float32),
            pltpu.VMEM((N_DEV - 1, SQ_SHARD, D_MODEL), jnp.float32),
            pltpu.SemaphoreType.DMA((N_DEV - 1,)),
            pltpu.SemaphoreType.DMA((N_DEV - 1,)),
            pltpu.SemaphoreType.DMA((N_DEV - 1,)),
            pltpu.SemaphoreType.DMA((N_DEV - 1,)),
            pltpu.SemaphoreType.DMA,
        ],
        compiler_params=pltpu.CompilerParams(collective_id=0),
    )(x_b, wq_b, k_l, v_l, wo_b)


# baseline (device time: 265018 ns/iter reference)
import functools

import jax
import jax.numpy as jnp
from jax import lax
from jax.experimental import pallas as pl
from jax.experimental.pallas import tpu as pltpu

N_DEV = 8
SQ = 2048
SQ_SHARD = 256
KV_USED = 2048
H_LOC = 8
DH = 128
D_MODEL = 1024
SCALE = 0.08838834764831843
NEG = -1e9


def _body(x_ref, wq_ref, k_ref, v_ref, wo_ref, out_ref,
          xg_ref, partial_ref, send_ref, comm_ref,
          ag_send_sems, ag_recv_sems, rs_send_sems, rs_recv_sems,
          copy_sem):
    i = lax.axis_index("i")
    right = lax.rem(i + 1, N_DEV)
    left = lax.rem(i + N_DEV - 1, N_DEV)

    barrier_sem = pltpu.get_barrier_semaphore()
    pl.semaphore_signal(barrier_sem, inc=1, device_id=(left,),
                        device_id_type=pl.DeviceIdType.MESH)
    pl.semaphore_signal(barrier_sem, inc=1, device_id=(right,),
                        device_id_type=pl.DeviceIdType.MESH)
    pl.semaphore_wait(barrier_sem, 2)

    own = pltpu.make_async_copy(x_ref, xg_ref.at[i], copy_sem)
    own.start()
    own.wait()

    for h in range(N_DEV - 1):
        o_send = lax.rem(i + N_DEV - h, N_DEV)
        rdma = pltpu.make_async_remote_copy(
            src_ref=xg_ref.at[o_send],
            dst_ref=xg_ref.at[o_send],
            send_sem=ag_send_sems.at[h],
            recv_sem=ag_recv_sems.at[h],
            device_id=(right,),
            device_id_type=pl.DeviceIdType.MESH,
        )
        rdma.start()
        rdma.wait()

    r_idx = lax.broadcasted_iota(jnp.int32, (SQ_SHARD, SQ_SHARD), 0)
    c_idx = lax.broadcasted_iota(jnp.int32, (SQ_SHARD, SQ_SHARD), 1)
    diag_keep = (c_idx // 64) <= (r_idx // 64)

    for o in range(N_DEV):
        L = SQ_SHARD * (o + 1)
        xq = xg_ref[o]

        def head_step(h, acc, L=L, xq=xq):
            q = jnp.dot(xq, wq_ref[h], preferred_element_type=jnp.float32)
            q = (q * SCALE).astype(jnp.bfloat16)
            k = k_ref[h, :L, :]
            s = lax.dot_general(q, k, (((1,), (1,)), ((), ())),
                                preferred_element_type=jnp.float32)
            s_diag = jnp.where(diag_keep, s[:, L - SQ_SHARD:], NEG)
            if L > SQ_SHARD:
                s = jnp.concatenate([s[:, :L - SQ_SHARD], s_diag], axis=1)
            else:
                s = s_diag
            m = jnp.max(s, axis=1, keepdims=True)
            w = jnp.exp(s - m)
            p = (w / jnp.sum(w, axis=1, keepdims=True)).astype(jnp.bfloat16)
            ctx = jnp.dot(p, v_ref[h, :L, :],
                          preferred_element_type=jnp.float32)
            return acc + jnp.dot(ctx.astype(jnp.bfloat16), wo_ref[h],
                                 preferred_element_type=jnp.float32)

        acc0 = jnp.zeros((SQ_SHARD, D_MODEL), jnp.float32)
        partial_ref[o] = lax.fori_loop(0, H_LOC, head_step, acc0)

    for t in range(N_DEV - 1):
        c_send = lax.rem(i + t + 1, N_DEV)
        if t == 0:
            send_ref[...] = partial_ref[c_send]
        else:
            send_ref[...] = comm_ref[t - 1] + partial_ref[c_send]
        rdma = pltpu.make_async_remote_copy(
            src_ref=send_ref,
            dst_ref=comm_ref.at[t],
            send_sem=rs_send_sems.at[t],
            recv_sem=rs_recv_sems.at[t],
            device_id=(left,),
            device_id_type=pl.DeviceIdType.MESH,
        )
        rdma.start()
        rdma.wait()

    out_ref[0] = comm_ref[N_DEV - 2] + partial_ref[i]


def kernel(x, Wq, K_ext, V_ext, Wo):
    i = lax.axis_index("i")

    x_b = x[0].astype(jnp.bfloat16)
    wq_b = Wq.reshape(D_MODEL, H_LOC, DH).transpose(1, 0, 2) \
             .astype(jnp.bfloat16)
    wo_b = Wo.reshape(H_LOC, DH, D_MODEL).astype(jnp.bfloat16)
    k_l = lax.dynamic_slice(
        K_ext, (0, 0, H_LOC * i, 0), (1, KV_USED, H_LOC, DH)
    )[0].transpose(1, 0, 2).astype(jnp.bfloat16)
    v_l = lax.dynamic_slice(
        V_ext, (0, 0, H_LOC * i, 0), (1, KV_USED, H_LOC, DH)
    )[0].transpose(1, 0, 2).astype(jnp.bfloat16)

    return pl.pallas_call(
        _body,
        out_shape=jax.ShapeDtypeStruct((1, SQ_SHARD, D_MODEL), jnp.float32),
        in_specs=[pl.BlockSpec(memory_space=pltpu.VMEM)] * 5,
        out_specs=pl.BlockSpec(memory_space=pltpu.VMEM),
        scratch_shapes=[
            pltpu.VMEM((N_DEV, SQ_SHARD, D_MODEL), jnp.bfloat16),
            pltpu.VMEM((N_DEV, SQ_SHARD, D_MODEL), jnp.float32),
            pltpu.VMEM((SQ_SHARD, D_MODEL), jnp.float32),
            pltpu.VMEM((N_DEV - 1, SQ_SHARD, D_MODEL), jnp.float32),
            pltpu.SemaphoreType.DMA((N_DEV - 1,)),
            pltpu.SemaphoreType.DMA((N_DEV - 1,)),
            pltpu.SemaphoreType.DMA((N_DEV - 1,)),
            pltpu.SemaphoreType.DMA((N_DEV - 1,)),
            pltpu.SemaphoreType.DMA,
        ],
        compiler_params=pltpu.CompilerParams(collective_id=0),
    )(x_b, wq_b, k_l, v_l, wo_b)


# device time: 237717 ns/iter; 1.1148x vs baseline; 1.1148x over previous
import functools

import jax
import jax.numpy as jnp
from jax import lax
from jax.experimental import pallas as pl
from jax.experimental.pallas import tpu as pltpu

N_DEV = 8
SQ_SHARD = 256
KV_USED = 2048
KV_BLK = 256
N_KVB = KV_USED // KV_BLK
H_LOC = 8
DH = 128
D_MODEL = 1024
SCALE = 0.08838834764831843
NEG = -1e9


def _body(x_ref, wq_ref, k_ref, v_ref, wo_ref, out_ref,
          xg_ref, comm_ref, send_ref, own_ref, s_ref, ctx_ref,
          ag_send_sems, ag_recv_sems, rs_send_sems, rs_recv_sems,
          copy_sem):
    i = lax.axis_index("i")
    right = lax.rem(i + 1, N_DEV)
    left = lax.rem(i + N_DEV - 1, N_DEV)

    barrier_sem = pltpu.get_barrier_semaphore()
    pl.semaphore_signal(barrier_sem, inc=1, device_id=(left,),
                        device_id_type=pl.DeviceIdType.MESH)
    pl.semaphore_signal(barrier_sem, inc=1, device_id=(right,),
                        device_id_type=pl.DeviceIdType.MESH)
    pl.semaphore_wait(barrier_sem, 2)

    r_idx = lax.broadcasted_iota(jnp.int32, (SQ_SHARD, KV_BLK), 0)
    c_idx = lax.broadcasted_iota(jnp.int32, (SQ_SHARD, KV_BLK), 1)
    diag_drop = (c_idx // 64) > (r_idx // 64)

    def compute_partial(o):
        xq = xg_ref[o]

        def head_step(h, acc):
            q = jnp.dot(xq, wq_ref[h], preferred_element_type=jnp.float32)
            q = (q * SCALE).astype(jnp.bfloat16)
            for kb in range(N_KVB):
                sl = slice(kb * KV_BLK, (kb + 1) * KV_BLK)

                @pl.when(kb <= o)
                def _(kb=kb, sl=sl, q=q, h=h):
                    s = lax.dot_general(
                        q, k_ref[h, sl, :], (((1,), (1,)), ((), ())),
                        preferred_element_type=jnp.float32)
                    s_ref[:, sl] = jnp.where((kb == o) & diag_drop, NEG, s)

                @pl.when(kb > o)
                def _(sl=sl):
                    s_ref[:, sl] = jnp.full((SQ_SHARD, KV_BLK), NEG,
                                            jnp.float32)

            s = s_ref[...]
            m = jnp.max(s, axis=1, keepdims=True)
            w = jnp.exp(s - m)
            p = (w / jnp.sum(w, axis=1, keepdims=True)).astype(jnp.bfloat16)
            ctx_ref[...] = jnp.zeros((SQ_SHARD, DH), jnp.float32)
            for kb in range(N_KVB):
                sl = slice(kb * KV_BLK, (kb + 1) * KV_BLK)

                @pl.when(kb <= o)
                def _(sl=sl, p=p, h=h):
                    ctx_ref[...] += jnp.dot(p[:, sl], v_ref[h, sl, :],
                                            preferred_element_type=jnp.float32)

            ctx = ctx_ref[...].astype(jnp.bfloat16)
            return acc + jnp.dot(ctx, wo_ref[h],
                                 preferred_element_type=jnp.float32)

        return lax.fori_loop(
            0, H_LOC, head_step,
            jnp.zeros((SQ_SHARD, D_MODEL), jnp.float32))

    def ag_desc(t, o):
        return pltpu.make_async_remote_copy(
            src_ref=xg_ref.at[o], dst_ref=xg_ref.at[o],
            send_sem=ag_send_sems.at[t], recv_sem=ag_recv_sems.at[t],
            device_id=(right,), device_id_type=pl.DeviceIdType.MESH)

    def rs_desc(t):
        return pltpu.make_async_remote_copy(
            src_ref=send_ref, dst_ref=comm_ref.at[t],
            send_sem=rs_send_sems.at[t], recv_sem=rs_recv_sems.at[t],
            device_id=(right,), device_id_type=pl.DeviceIdType.MESH)

    own = pltpu.make_async_copy(x_ref, xg_ref.at[i], copy_sem)
    own.start()
    own.wait()
    ag_desc(0, i).start()
    own_ref[...] = compute_partial(i)

    def pipe_step(t, carry):
        o = lax.rem(i + N_DEV - 1 - t, N_DEV)
        ag_desc(t, o).wait_recv()

        @pl.when(t < N_DEV - 2)
        def _():
            ag_desc(jnp.minimum(t + 1, N_DEV - 2), o).start()

        acc = compute_partial(o)

        @pl.when(t == 0)
        def _():
            send_ref[...] = acc.astype(jnp.bfloat16)

        @pl.when(t > 0)
        def _():
            tm1 = jnp.maximum(t - 1, 0)
            prev = rs_desc(tm1)
            prev.wait_send()
            prev.wait_recv()
            send_ref[...] = (acc + comm_ref[tm1].astype(jnp.float32)
                             ).astype(jnp.bfloat16)

        rs_desc(t).start()
        return carry

    lax.fori_loop(0, N_DEV - 1, pipe_step, 0)

    last = rs_desc(N_DEV - 2)
    last.wait_send()
    last.wait_recv()
    out_ref[0] = comm_ref[N_DEV - 2].astype(jnp.float32) + own_ref[...]

    for h in range(N_DEV - 1):
        ag_desc(h, 0).wait_send()

    @functools.partial(pl.run_scoped, exit_sem=pltpu.SemaphoreType.REGULAR)
    def _(exit_sem):
        pl.semaphore_signal(exit_sem, inc=1, device_id=(left,),
                            device_id_type=pl.DeviceIdType.MESH)
        pl.semaphore_signal(exit_sem, inc=1, device_id=(right,),
                            device_id_type=pl.DeviceIdType.MESH)
        pl.semaphore_wait(exit_sem, 2)


def kernel(x, Wq, K_ext, V_ext, Wo):
    i = lax.axis_index("i")

    x_b = x[0].astype(jnp.bfloat16)
    wq_b = Wq.reshape(D_MODEL, H_LOC, DH).transpose(1, 0, 2) \
             .astype(jnp.bfloat16)
    wo_b = Wo.reshape(H_LOC, DH, D_MODEL).astype(jnp.bfloat16)
    k_l = lax.dynamic_slice(
        K_ext, (0, 0, H_LOC * i, 0), (1, KV_USED, H_LOC, DH)
    )[0].transpose(1, 0, 2).astype(jnp.bfloat16)
    v_l = lax.dynamic_slice(
        V_ext, (0, 0, H_LOC * i, 0), (1, KV_USED, H_LOC, DH)
    )[0].transpose(1, 0, 2).astype(jnp.bfloat16)

    return pl.pallas_call(
        _body,
        out_shape=jax.ShapeDtypeStruct((1, SQ_SHARD, D_MODEL), jnp.float32),
        in_specs=[pl.BlockSpec(memory_space=pltpu.VMEM)] * 5,
        out_specs=pl.BlockSpec(memory_space=pltpu.VMEM),
        scratch_shapes=[
            pltpu.VMEM((N_DEV, SQ_SHARD, D_MODEL), jnp.bfloat16),
            pltpu.VMEM((N_DEV - 1, SQ_SHARD, D_MODEL), jnp.bfloat16),
            pltpu.VMEM((SQ_SHARD, D_MODEL), jnp.bfloat16),
            pltpu.VMEM((SQ_SHARD, D_MODEL), jnp.float32),
            pltpu.VMEM((SQ_SHARD, KV_USED), jnp.float32),
            pltpu.VMEM((SQ_SHARD, DH), jnp.float32),
            pltpu.SemaphoreType.DMA((N_DEV - 1,)),
            pltpu.SemaphoreType.DMA((N_DEV - 1,)),
            pltpu.SemaphoreType.DMA((N_DEV - 1,)),
            pltpu.SemaphoreType.DMA((N_DEV - 1,)),
            pltpu.SemaphoreType.DMA,
        ],
        compiler_params=pltpu.CompilerParams(collective_id=0),
    )(x_b, wq_b, k_l, v_l, wo_b)
